# baseline (speedup 1.0000x reference)
import functools

import jax
import jax.numpy as jnp
from jax import lax
from jax.experimental import pallas as pl
from jax.experimental.pallas import tpu as pltpu
from jax.experimental.pallas import tpu_sc as plsc

WIN = 256
NUM_HEADS = 16
TABLE_ROWS = 961
NUM_WORKERS = 32
ROWS_PER_WORKER = WIN // NUM_WORKERS
LANES = 16


def _sc_gather_body(table_hbm, idx_hbm, out_hbm, idx_v, table_v, out_v,
                    sem_idx, sem_tab, sem_out):
    wid = lax.axis_index("s") * 2 + lax.axis_index("c")
    row0 = wid * ROWS_PER_WORKER
    cp_idx = pltpu.async_copy(
        idx_hbm.at[pl.ds(row0, ROWS_PER_WORKER)], idx_v, sem_idx)
    cp_tab = pltpu.async_copy(table_hbm, table_v, sem_tab)
    cp_idx.wait()
    cp_tab.wait()

    groups_per_row = WIN // LANES

    def group(g):
        r = g // groups_per_row
        c = (g % groups_per_row) * LANES
        iv = idx_v[r, pl.ds(c, LANES)]
        vals = [plsc.load_gather(table_v, [iv + h * TABLE_ROWS])
                for h in range(NUM_HEADS)]
        for h in range(NUM_HEADS):
            out_v[h, r, pl.ds(c, LANES)] = vals[h]

    ngroups = ROWS_PER_WORKER * groups_per_row
    half_rows = ROWS_PER_WORKER // 2
    plsc.parallel_loop(0, ngroups // 2, unroll=4)(group)
    cp_out = pltpu.async_copy(
        out_v.at[:, pl.ds(0, half_rows)],
        out_hbm.at[:, pl.ds(row0, half_rows)], sem_out)
    plsc.parallel_loop(ngroups // 2, ngroups, unroll=4)(group)
    cp_out.wait()
    pltpu.sync_copy(out_v.at[:, pl.ds(half_rows, half_rows)],
                    out_hbm.at[:, pl.ds(row0 + half_rows, half_rows)])


_sc_gather = functools.partial(
    pl.kernel,
    out_type=jax.ShapeDtypeStruct((NUM_HEADS, WIN, WIN), jnp.float32),
    mesh=plsc.VectorSubcoreMesh(core_axis_name="c", subcore_axis_name="s"),
    compiler_params=pltpu.CompilerParams(needs_layout_passes=False),
    scratch_types=[
        pltpu.VMEM((ROWS_PER_WORKER, WIN), jnp.int32),
        pltpu.VMEM((TABLE_ROWS * NUM_HEADS,), jnp.float32),
        pltpu.VMEM((NUM_HEADS, ROWS_PER_WORKER, WIN), jnp.float32),
        pltpu.SemaphoreType.DMA,
        pltpu.SemaphoreType.DMA,
        pltpu.SemaphoreType.DMA,
    ],
)(_sc_gather_body)


def _add_body(x_ref, b_ref, o_ref):
    o_ref[...] = x_ref[...] + b_ref[...]


def kernel(x, relative_position_bias_table, relative_position_index):
    batch, heads, area, _ = x.shape
    idx32 = relative_position_index.astype(jnp.int32)
    table_t = relative_position_bias_table.T.reshape(-1)
    bias_t = _sc_gather(table_t, idx32)

    bb = 2
    out = pl.pallas_call(
        _add_body,
        grid=(batch // bb,),
        in_specs=[
            pl.BlockSpec((bb, heads, area, area), lambda b: (b, 0, 0, 0)),
            pl.BlockSpec((heads, area, area), lambda b: (0, 0, 0)),
        ],
        out_specs=pl.BlockSpec((bb, heads, area, area), lambda b: (b, 0, 0, 0)),
        out_shape=jax.ShapeDtypeStruct(x.shape, x.dtype),
    )(x, bias_t)
    return out


# speedup vs baseline: 1.0103x; 1.0103x over previous
import functools

import jax
import jax.numpy as jnp
from jax import lax
from jax.experimental import pallas as pl
from jax.experimental.pallas import tpu as pltpu
from jax.experimental.pallas import tpu_sc as plsc

WIN = 256
NUM_HEADS = 16
TABLE_ROWS = 961
NUM_WORKERS = 32
ROWS_PER_WORKER = WIN // NUM_WORKERS
LANES = 16


def _sc_gather_body(table_hbm, idx_hbm, out_hbm, idx_v, table_v, out_v,
                    sem_idx, sem_tab, sem_out):
    wid = lax.axis_index("s") * 2 + lax.axis_index("c")
    row0 = wid * ROWS_PER_WORKER
    cp_idx = pltpu.async_copy(
        idx_hbm.at[pl.ds(row0, ROWS_PER_WORKER)], idx_v, sem_idx)
    cp_tab = pltpu.async_copy(table_hbm, table_v, sem_tab)
    cp_idx.wait()
    cp_tab.wait()

    groups_per_row = WIN // LANES
    hsplat = [jnp.full((LANES,), h, jnp.int32) for h in range(NUM_HEADS)]

    def group(g):
        r = g // groups_per_row
        c = (g % groups_per_row) * LANES
        iv = idx_v[r, pl.ds(c, LANES)]
        vals = [plsc.load_gather(table_v, [hsplat[h], iv])
                for h in range(NUM_HEADS)]
        for h in range(NUM_HEADS):
            out_v[h, r, pl.ds(c, LANES)] = vals[h]

    ngroups = ROWS_PER_WORKER * groups_per_row
    half_rows = ROWS_PER_WORKER // 2
    plsc.parallel_loop(0, ngroups // 2, unroll=2)(group)
    cp_out = pltpu.async_copy(
        out_v.at[:, pl.ds(0, half_rows)],
        out_hbm.at[:, pl.ds(row0, half_rows)], sem_out)
    plsc.parallel_loop(ngroups // 2, ngroups, unroll=2)(group)
    cp_out.wait()
    pltpu.sync_copy(out_v.at[:, pl.ds(half_rows, half_rows)],
                    out_hbm.at[:, pl.ds(row0 + half_rows, half_rows)])


_sc_gather = functools.partial(
    pl.kernel,
    out_type=jax.ShapeDtypeStruct((NUM_HEADS, WIN, WIN), jnp.float32),
    mesh=plsc.VectorSubcoreMesh(core_axis_name="c", subcore_axis_name="s"),
    compiler_params=pltpu.CompilerParams(needs_layout_passes=False),
    scratch_types=[
        pltpu.VMEM((ROWS_PER_WORKER, WIN), jnp.int32),
        pltpu.VMEM((NUM_HEADS, TABLE_ROWS), jnp.float32),
        pltpu.VMEM((NUM_HEADS, ROWS_PER_WORKER, WIN), jnp.float32),
        pltpu.SemaphoreType.DMA,
        pltpu.SemaphoreType.DMA,
        pltpu.SemaphoreType.DMA,
    ],
)(_sc_gather_body)


def _add_body(x_ref, b_ref, o_ref):
    o_ref[...] = x_ref[...] + b_ref[...]


def kernel(x, relative_position_bias_table, relative_position_index):
    batch, heads, area, _ = x.shape
    idx32 = relative_position_index.astype(jnp.int32)
    bias_t = _sc_gather(relative_position_bias_table.T, idx32)

    bb = 2
    out = pl.pallas_call(
        _add_body,
        grid=(batch // bb,),
        in_specs=[
            pl.BlockSpec((bb, heads, area, area), lambda b: (b, 0, 0, 0)),
            pl.BlockSpec((heads, area, area), lambda b: (0, 0, 0)),
        ],
        out_specs=pl.BlockSpec((bb, heads, area, area), lambda b: (b, 0, 0, 0)),
        out_shape=jax.ShapeDtypeStruct(x.shape, x.dtype),
    )(x, bias_t)
    return out


# speedup vs baseline: 1.0186x; 1.0082x over previous
import functools

import jax
import jax.numpy as jnp
from jax import lax
from jax.experimental import pallas as pl
from jax.experimental.pallas import tpu as pltpu
from jax.experimental.pallas import tpu_sc as plsc

WIN = 256
NUM_HEADS = 16
TABLE_ROWS = 961
NUM_WORKERS = 32
ROWS_PER_WORKER = WIN // NUM_WORKERS
LANES = 16


def _sc_gather_body(table_hbm, idx_hbm, out_hbm, idx_v, table_v, out_v,
                    sem_idx, sem_tab, sem_out):
    wid = lax.axis_index("s") * 2 + lax.axis_index("c")
    row0 = wid * ROWS_PER_WORKER
    cp_idx = pltpu.async_copy(
        idx_hbm.at[pl.ds(row0, ROWS_PER_WORKER)], idx_v, sem_idx)
    cp_tab = pltpu.async_copy(table_hbm, table_v, sem_tab)
    cp_idx.wait()
    cp_tab.wait()

    groups_per_row = WIN // LANES

    def group(g):
        r = g // groups_per_row
        c = (g % groups_per_row) * LANES
        iv = idx_v[r, pl.ds(c, LANES)]
        vals = [plsc.load_gather(table_v, [iv + h * TABLE_ROWS])
                for h in range(NUM_HEADS)]
        for h in range(NUM_HEADS):
            out_v[h, r, pl.ds(c, LANES)] = vals[h]

    ngroups = ROWS_PER_WORKER * groups_per_row
    half_rows = ROWS_PER_WORKER // 2
    plsc.parallel_loop(0, ngroups // 2, unroll=2)(group)
    cp_out = pltpu.async_copy(
        out_v.at[:, pl.ds(0, half_rows)],
        out_hbm.at[:, pl.ds(row0, half_rows)], sem_out)
    plsc.parallel_loop(ngroups // 2, ngroups, unroll=2)(group)
    cp_out.wait()
    pltpu.sync_copy(out_v.at[:, pl.ds(half_rows, half_rows)],
                    out_hbm.at[:, pl.ds(row0 + half_rows, half_rows)])


_sc_gather = functools.partial(
    pl.kernel,
    out_type=jax.ShapeDtypeStruct((NUM_HEADS, WIN, WIN), jnp.float32),
    mesh=plsc.VectorSubcoreMesh(core_axis_name="c", subcore_axis_name="s"),
    compiler_params=pltpu.CompilerParams(needs_layout_passes=False),
    scratch_types=[
        pltpu.VMEM((ROWS_PER_WORKER, WIN), jnp.int32),
        pltpu.VMEM((TABLE_ROWS * NUM_HEADS,), jnp.float32),
        pltpu.VMEM((NUM_HEADS, ROWS_PER_WORKER, WIN), jnp.float32),
        pltpu.SemaphoreType.DMA,
        pltpu.SemaphoreType.DMA,
        pltpu.SemaphoreType.DMA,
    ],
)(_sc_gather_body)


def _add_body(x_ref, b_ref, o_ref):
    o_ref[...] = x_ref[...] + b_ref[...]


def kernel(x, relative_position_bias_table, relative_position_index):
    batch, heads, area, _ = x.shape
    idx32 = relative_position_index.astype(jnp.int32)
    bias_t = _sc_gather(relative_position_bias_table.T.reshape(-1), idx32)

    bb = 2
    out = pl.pallas_call(
        _add_body,
        grid=(batch // bb,),
        in_specs=[
            pl.BlockSpec((bb, heads, area, area), lambda b: (b, 0, 0, 0)),
            pl.BlockSpec((heads, area, area), lambda b: (0, 0, 0)),
        ],
        out_specs=pl.BlockSpec((bb, heads, area, area), lambda b: (b, 0, 0, 0)),
        out_shape=jax.ShapeDtypeStruct(x.shape, x.dtype),
    )(x, bias_t)
    return out
